# Initial kernel scaffold; baseline (speedup 1.0000x reference)
import jax
import jax.numpy as jnp
from jax.experimental import pallas as pl


def kernel(x, edge_index, batch, W1, b1, W2, b2):
    raise NotImplementedError("write your pallas kernel here")


# speedup vs baseline: 13.3569x; 13.3569x over previous
import functools

import jax
import jax.numpy as jnp
from jax import lax
from jax.experimental import pallas as pl
from jax.experimental.pallas import tpu as pltpu
from jax.experimental.pallas import tpu_sc as plsc

N = 10000
E = 320000
D = 128

NC = 2
NS = 16
NW = NC * NS
EPW = E // NW
CH = 80
NCH = EPW // CH
RPT = N // NS
ZR = 125

_mesh = plsc.VectorSubcoreMesh(core_axis_name="c", subcore_axis_name="s")
_sc_params = pltpu.CompilerParams(
    needs_layout_passes=False, use_tc_tiling_on_sc=False
)


@functools.partial(
    pl.kernel,
    mesh=_mesh,
    out_type=jax.ShapeDtypeStruct((NW, N), jnp.float32),
    compiler_params=_sc_params,
    scratch_types=[
        pltpu.VMEM((N,), jnp.float32),
        pltpu.VMEM((CH,), jnp.int32),
    ],
)
def _deg_pass(dst_hbm, out_hbm, deg_l, idx_b):
    cid = lax.axis_index("c")
    sid = lax.axis_index("s")
    w = sid * NC + cid
    z16 = jnp.zeros((16,), jnp.float32)
    ones16 = jnp.ones((16,), jnp.float32)

    def zloop(i, c):
        deg_l[pl.ds(i * 16, 16)] = z16
        return c

    lax.fori_loop(0, N // 16, zloop, 0)

    def cloop(i, c):
        pltpu.sync_copy(dst_hbm.at[pl.ds(w * EPW + i * CH, CH)], idx_b)
        for j in range(CH // 16):
            idx = idx_b[pl.ds(j * 16, 16)]
            plsc.addupdate_scatter(deg_l, [idx], ones16)
        return c

    lax.fori_loop(0, NCH, cloop, 0)
    pltpu.sync_copy(deg_l, out_hbm.at[w])


@functools.partial(
    pl.kernel,
    mesh=_mesh,
    out_type=jax.ShapeDtypeStruct((NC, N, D), jnp.float32),
    compiler_params=_sc_params,
    scratch_types=[
        pltpu.VMEM((CH,), jnp.int32),
        pltpu.VMEM((CH,), jnp.int32),
        pltpu.VMEM((CH, D), jnp.float32),
        pltpu.VMEM((ZR, D), jnp.float32),
        pltpu.VMEM_SHARED((N, D), jnp.float32),
        pltpu.SemaphoreType.DMA,
    ],
)
def _edge_pass(g_hbm, src_hbm, dst_hbm, out_hbm, sidx, didx, rows, zbuf, acc, sem):
    cid = lax.axis_index("c")
    sid = lax.axis_index("s")
    w = sid * NC + cid
    z16 = jnp.zeros((16,), jnp.float32)

    def zrow(i, c):
        def zcol(j, c2):
            zbuf[i, pl.ds(j * 16, 16)] = z16
            return c2
        return lax.fori_loop(0, D // 16, zcol, c)

    lax.fori_loop(0, ZR, zrow, 0)
    for r in range(RPT // ZR):
        pltpu.sync_copy(zbuf, acc.at[pl.ds(sid * RPT + r * ZR, ZR)])
    plsc.subcore_barrier()

    def eloop(i, c):
        base = w * EPW + i * CH
        pltpu.sync_copy(src_hbm.at[pl.ds(base, CH)], sidx)
        pltpu.sync_copy(dst_hbm.at[pl.ds(base, CH)], didx)
        pltpu.async_copy(g_hbm.at[sidx], rows, sem).wait()
        pltpu.sync_copy(rows, acc.at[didx], add=True)
        return c

    lax.fori_loop(0, NCH, eloop, 0)
    plsc.subcore_barrier()
    pltpu.sync_copy(
        acc.at[pl.ds(sid * RPT, RPT)],
        out_hbm.at[cid, pl.ds(sid * RPT, RPT)],
    )


_BLK = 1000
_GRID = N // _BLK


def _tc1_body(degs_ref, x_ref, w_ref, g_ref, dinv_ref):
    dinv = lax.rsqrt(jnp.sum(degs_ref[...], axis=1, keepdims=True) + 1.0)
    h = jnp.dot(x_ref[...], w_ref[...], preferred_element_type=jnp.float32)
    g_ref[...] = h * dinv
    dinv_ref[...] = dinv


def _tc2_body(dinv_ref, s_ref, g1_ref, b1_ref, w2_ref, g2_ref):
    dinv = dinv_ref[...]
    s = s_ref[0] + s_ref[1] + g1_ref[...]
    h = jnp.maximum(s * dinv + b1_ref[...], 0.0)
    g2_ref[...] = jnp.dot(h, w2_ref[...], preferred_element_type=jnp.float32) * dinv


def _tc3_body(dinv_ref, s_ref, g2_ref, b2_ref, o_ref):
    z = (s_ref[0] + s_ref[1] + g2_ref[...]) * dinv_ref[...] + b2_ref[...]
    m = jnp.max(z, axis=1, keepdims=True)
    zs = z - m
    o_ref[...] = zs - jnp.log(jnp.sum(jnp.exp(zs), axis=1, keepdims=True))


_deg_spec = pl.BlockSpec((_BLK, NW), lambda i: (i, 0))
_dinv_spec = pl.BlockSpec((_BLK, 1), lambda i: (i, 0))
_row_spec = pl.BlockSpec((_BLK, D), lambda i: (i, 0))
_s_spec = pl.BlockSpec((NC, _BLK, D), lambda i: (0, i, 0))
_w_spec = pl.BlockSpec((D, D), lambda i: (0, 0))
_b_spec = pl.BlockSpec((1, D), lambda i: (0, 0))

_tc1 = pl.pallas_call(
    _tc1_body,
    grid=(_GRID,),
    in_specs=[_deg_spec, _row_spec, _w_spec],
    out_specs=[_row_spec, _dinv_spec],
    out_shape=[
        jax.ShapeDtypeStruct((N, D), jnp.float32),
        jax.ShapeDtypeStruct((N, 1), jnp.float32),
    ],
)

_tc2 = pl.pallas_call(
    _tc2_body,
    grid=(_GRID,),
    in_specs=[_dinv_spec, _s_spec, _row_spec, _b_spec, _w_spec],
    out_specs=_row_spec,
    out_shape=jax.ShapeDtypeStruct((N, D), jnp.float32),
)

_tc3 = pl.pallas_call(
    _tc3_body,
    grid=(_GRID,),
    in_specs=[_dinv_spec, _s_spec, _row_spec, _b_spec],
    out_specs=_row_spec,
    out_shape=jax.ShapeDtypeStruct((N, D), jnp.float32),
)


def kernel(x, edge_index, batch, W1, b1, W2, b2):
    src = edge_index[0]
    dst = edge_index[1]
    degs = _deg_pass(dst)
    g1, dinv = _tc1(degs.T, x, W1)
    s1 = _edge_pass(g1, src, dst)
    g2 = _tc2(dinv, s1, g1, b1.reshape(1, D), W2)
    s2 = _edge_pass(g2, src, dst)
    return _tc3(dinv, s2, g2, b2.reshape(1, D))


# speedup vs baseline: 35.4623x; 2.6550x over previous
import functools

import jax
import jax.numpy as jnp
from jax import lax
from jax.experimental import pallas as pl
from jax.experimental.pallas import tpu as pltpu
from jax.experimental.pallas import tpu_sc as plsc

N = 10000
E = 320000
D = 128

NC = 2
NS = 16
NW = NC * NS
EPW = E // NW
CH = 80
NCH = EPW // CH
RPT = N // NS
ZR = 125

_mesh = plsc.VectorSubcoreMesh(core_axis_name="c", subcore_axis_name="s")
_sc_params = pltpu.CompilerParams(
    needs_layout_passes=False, use_tc_tiling_on_sc=False
)


@functools.partial(
    pl.kernel,
    mesh=_mesh,
    out_type=jax.ShapeDtypeStruct((NW, N), jnp.float32),
    compiler_params=_sc_params,
    scratch_types=[
        pltpu.VMEM((N,), jnp.float32),
        pltpu.VMEM((EPW,), jnp.int32),
    ],
)
def _deg_pass(dst_hbm, out_hbm, deg_l, idx_b):
    cid = lax.axis_index("c")
    sid = lax.axis_index("s")
    w = sid * NC + cid
    z16 = jnp.zeros((16,), jnp.float32)
    ones16 = jnp.ones((16,), jnp.float32)

    def zloop(i, c):
        deg_l[pl.ds(i * 16, 16)] = z16
        return c

    lax.fori_loop(0, N // 16, zloop, 0)
    pltpu.sync_copy(dst_hbm.at[w], idx_b)

    def cloop(i, c):
        idx = idx_b[pl.ds(i * 16, 16)]
        plsc.addupdate_scatter(deg_l, [idx], ones16)
        return c

    lax.fori_loop(0, EPW // 16, cloop, 0)
    pltpu.sync_copy(deg_l, out_hbm.at[w])


NB = 3
PK = 2


@functools.partial(
    pl.kernel,
    mesh=_mesh,
    out_type=jax.ShapeDtypeStruct((NC, N, D), jnp.float32),
    compiler_params=_sc_params,
    scratch_types=[
        pltpu.VMEM((NCH, CH), jnp.int32),
        pltpu.VMEM((NCH, CH), jnp.int32),
        pltpu.VMEM((NB, CH, D), jnp.float32),
        pltpu.VMEM_SHARED((N, D), jnp.float32),
        pltpu.SemaphoreType.DMA((NB,)),
        pltpu.SemaphoreType.DMA((NB,)),
    ],
)
def _edge_pass(g_hbm, src_hbm, dst_hbm, out_hbm, sidx, didx, rows, acc,
               sem_g, sem_s):
    cid = lax.axis_index("c")
    sid = lax.axis_index("s")
    w = sid * NC + cid
    z16 = jnp.zeros((16,), jnp.float32)

    pltpu.sync_copy(src_hbm.at[w], sidx)
    pltpu.sync_copy(dst_hbm.at[w], didx)

    def zrow(i, c):
        def zcol(j, c2):
            rows[0, i, pl.ds(j * 16, 16)] = z16
            return c2
        return lax.fori_loop(0, D // 16, zcol, c)

    lax.fori_loop(0, CH, zrow, 0)
    for r in range(RPT // CH):
        pltpu.sync_copy(rows.at[0], acc.at[pl.ds(sid * RPT + r * CH, CH)])
    rem = RPT - (RPT // CH) * CH
    if rem:
        pltpu.sync_copy(
            rows.at[0, pl.ds(0, rem)],
            acc.at[pl.ds(sid * RPT + (RPT // CH) * CH, rem)],
        )
    plsc.subcore_barrier()

    for j in range(PK):
        pltpu.async_copy(g_hbm.at[sidx.at[j]], rows.at[j], sem_g.at[j])

    def eloop(i, c):
        b = lax.rem(i, NB)
        pltpu.make_async_copy(g_hbm.at[pl.ds(0, CH)], rows.at[b],
                              sem_g.at[b]).wait()
        pltpu.async_copy(rows.at[b], acc.at[didx.at[i]], sem_s.at[b], add=True)

        nxt = i + PK
        bn = lax.rem(nxt, NB)

        @pl.when(nxt < NCH)
        def _():
            @pl.when(nxt >= NB)
            def _():
                pltpu.make_async_copy(rows.at[bn], acc.at[pl.ds(0, CH)],
                                      sem_s.at[bn]).wait()
            pltpu.async_copy(g_hbm.at[sidx.at[nxt]], rows.at[bn],
                             sem_g.at[bn])
        return c

    lax.fori_loop(0, NCH, eloop, 0)
    for j in range(NB):
        pltpu.make_async_copy(rows.at[j], acc.at[pl.ds(0, CH)],
                              sem_s.at[j]).wait()
    plsc.subcore_barrier()
    pltpu.sync_copy(
        acc.at[pl.ds(sid * RPT, RPT)],
        out_hbm.at[cid, pl.ds(sid * RPT, RPT)],
    )


_BLK = 1000
_GRID = N // _BLK


def _tc1_body(degs_ref, x_ref, w_ref, g_ref, dinv_ref):
    dinv = lax.rsqrt(jnp.sum(degs_ref[...], axis=1, keepdims=True) + 1.0)
    h = jnp.dot(x_ref[...], w_ref[...], preferred_element_type=jnp.float32)
    g_ref[...] = h * dinv
    dinv_ref[...] = dinv


def _tc2_body(dinv_ref, s_ref, g1_ref, b1_ref, w2_ref, g2_ref):
    dinv = dinv_ref[...]
    s = s_ref[0] + s_ref[1] + g1_ref[...]
    h = jnp.maximum(s * dinv + b1_ref[...], 0.0)
    g2_ref[...] = jnp.dot(h, w2_ref[...], preferred_element_type=jnp.float32) * dinv


def _tc3_body(dinv_ref, s_ref, g2_ref, b2_ref, o_ref):
    z = (s_ref[0] + s_ref[1] + g2_ref[...]) * dinv_ref[...] + b2_ref[...]
    m = jnp.max(z, axis=1, keepdims=True)
    zs = z - m
    o_ref[...] = zs - jnp.log(jnp.sum(jnp.exp(zs), axis=1, keepdims=True))


_deg_spec = pl.BlockSpec((_BLK, NW), lambda i: (i, 0))
_dinv_spec = pl.BlockSpec((_BLK, 1), lambda i: (i, 0))
_row_spec = pl.BlockSpec((_BLK, D), lambda i: (i, 0))
_s_spec = pl.BlockSpec((NC, _BLK, D), lambda i: (0, i, 0))
_w_spec = pl.BlockSpec((D, D), lambda i: (0, 0))
_b_spec = pl.BlockSpec((1, D), lambda i: (0, 0))

_tc1 = pl.pallas_call(
    _tc1_body,
    grid=(_GRID,),
    in_specs=[_deg_spec, _row_spec, _w_spec],
    out_specs=[_row_spec, _dinv_spec],
    out_shape=[
        jax.ShapeDtypeStruct((N, D), jnp.float32),
        jax.ShapeDtypeStruct((N, 1), jnp.float32),
    ],
)

_tc2 = pl.pallas_call(
    _tc2_body,
    grid=(_GRID,),
    in_specs=[_dinv_spec, _s_spec, _row_spec, _b_spec, _w_spec],
    out_specs=_row_spec,
    out_shape=jax.ShapeDtypeStruct((N, D), jnp.float32),
)

_tc3 = pl.pallas_call(
    _tc3_body,
    grid=(_GRID,),
    in_specs=[_dinv_spec, _s_spec, _row_spec, _b_spec],
    out_specs=_row_spec,
    out_shape=jax.ShapeDtypeStruct((N, D), jnp.float32),
)


def kernel(x, edge_index, batch, W1, b1, W2, b2):
    e3 = edge_index.reshape(2, NW, NCH, CH)
    src = e3[0]
    dst = e3[1]
    degs = _deg_pass(edge_index[1].reshape(NW, EPW))
    g1, dinv = _tc1(degs.T, x, W1)
    s1 = _edge_pass(g1, src, dst)
    g2 = _tc2(dinv, s1, g1, b1.reshape(1, D), W2)
    s2 = _edge_pass(g2, src, dst)
    return _tc3(dinv, s2, g2, b2.reshape(1, D))


# speedup vs baseline: 38.1369x; 1.0754x over previous
import functools

import jax
import jax.numpy as jnp
from jax import lax
from jax.experimental import pallas as pl
from jax.experimental.pallas import tpu as pltpu
from jax.experimental.pallas import tpu_sc as plsc

N = 10000
E = 320000
D = 128

NC = 2
NS = 16
NW = NC * NS
EPW = E // NW
CH = 80
NCH = EPW // CH
RPT = N // NS
ZR = 125

_mesh = plsc.VectorSubcoreMesh(core_axis_name="c", subcore_axis_name="s")
_sc_params = pltpu.CompilerParams(
    needs_layout_passes=False, use_tc_tiling_on_sc=False
)


@functools.partial(
    pl.kernel,
    mesh=_mesh,
    out_type=jax.ShapeDtypeStruct((NW, N), jnp.float32),
    compiler_params=_sc_params,
    scratch_types=[
        pltpu.VMEM((N,), jnp.float32),
        pltpu.VMEM((EPW,), jnp.int32),
    ],
)
def _deg_pass(e2_hbm, out_hbm, deg_l, idx_b):
    cid = lax.axis_index("c")
    sid = lax.axis_index("s")
    w = sid * NC + cid
    z16 = jnp.zeros((16,), jnp.float32)
    ones16 = jnp.ones((16,), jnp.float32)

    def zloop(i, c):
        deg_l[pl.ds(i * 16, 16)] = z16
        return c

    lax.fori_loop(0, N // 16, zloop, 0)
    pltpu.sync_copy(e2_hbm.at[1, w], idx_b)

    def cloop(i, c):
        idx = idx_b[pl.ds(i * 16, 16)]
        plsc.addupdate_scatter(deg_l, [idx], ones16)
        return c

    lax.fori_loop(0, EPW // 16, cloop, 0)
    pltpu.sync_copy(deg_l, out_hbm.at[w])


NB = 3
PK = 2


@functools.partial(
    pl.kernel,
    mesh=_mesh,
    out_type=jax.ShapeDtypeStruct((NC, N, D), jnp.float32),
    compiler_params=_sc_params,
    scratch_types=[
        pltpu.VMEM((NCH, CH), jnp.int32),
        pltpu.VMEM((NCH, CH), jnp.int32),
        pltpu.VMEM((NB, CH, D), jnp.float32),
        pltpu.VMEM_SHARED((N, D), jnp.float32),
        pltpu.SemaphoreType.DMA((NB,)),
        pltpu.SemaphoreType.DMA((NB,)),
    ],
)
def _edge_pass(g_hbm, e4_hbm, out_hbm, sidx, didx, rows, acc,
               sem_g, sem_s):
    cid = lax.axis_index("c")
    sid = lax.axis_index("s")
    w = sid * NC + cid
    z16 = jnp.zeros((16,), jnp.float32)

    pltpu.sync_copy(e4_hbm.at[0, w], sidx)
    pltpu.sync_copy(e4_hbm.at[1, w], didx)

    def zrow(i, c):
        def zcol(j, c2):
            rows[0, i, pl.ds(j * 16, 16)] = z16
            return c2
        return lax.fori_loop(0, D // 16, zcol, c)

    lax.fori_loop(0, CH, zrow, 0)
    for r in range(RPT // CH):
        pltpu.sync_copy(rows.at[0], acc.at[pl.ds(sid * RPT + r * CH, CH)])
    rem = RPT - (RPT // CH) * CH
    if rem:
        pltpu.sync_copy(
            rows.at[0, pl.ds(0, rem)],
            acc.at[pl.ds(sid * RPT + (RPT // CH) * CH, rem)],
        )
    plsc.subcore_barrier()

    for j in range(PK):
        pltpu.async_copy(g_hbm.at[sidx.at[j]], rows.at[j], sem_g.at[j])

    def eloop(i, c):
        b = lax.rem(i, NB)
        pltpu.make_async_copy(g_hbm.at[pl.ds(0, CH)], rows.at[b],
                              sem_g.at[b]).wait()
        pltpu.async_copy(rows.at[b], acc.at[didx.at[i]], sem_s.at[b], add=True)

        nxt = i + PK
        bn = lax.rem(nxt, NB)

        @pl.when(nxt < NCH)
        def _():
            @pl.when(nxt >= NB)
            def _():
                pltpu.make_async_copy(rows.at[bn], acc.at[pl.ds(0, CH)],
                                      sem_s.at[bn]).wait()
            pltpu.async_copy(g_hbm.at[sidx.at[nxt]], rows.at[bn],
                             sem_g.at[bn])
        return c

    lax.fori_loop(0, NCH, eloop, 0)
    for j in range(NB):
        pltpu.make_async_copy(rows.at[j], acc.at[pl.ds(0, CH)],
                              sem_s.at[j]).wait()
    plsc.subcore_barrier()
    pltpu.sync_copy(
        acc.at[pl.ds(sid * RPT, RPT)],
        out_hbm.at[cid, pl.ds(sid * RPT, RPT)],
    )


_BLK = 2000
_GRID = N // _BLK


def _tc1_body(degs_ref, x_ref, w_ref, g_ref, dinv_ref):
    dinv = lax.rsqrt(jnp.sum(degs_ref[...], axis=1, keepdims=True) + 1.0)
    h = jnp.dot(x_ref[...], w_ref[...], preferred_element_type=jnp.float32)
    g_ref[...] = h * dinv
    dinv_ref[...] = dinv


def _tc2_body(dinv_ref, s_ref, g1_ref, b1_ref, w2_ref, g2_ref):
    dinv = dinv_ref[...]
    s = s_ref[0] + s_ref[1] + g1_ref[...]
    h = jnp.maximum(s * dinv + b1_ref[...], 0.0)
    g2_ref[...] = jnp.dot(h, w2_ref[...], preferred_element_type=jnp.float32) * dinv


def _tc3_body(dinv_ref, s_ref, g2_ref, b2_ref, o_ref):
    z = (s_ref[0] + s_ref[1] + g2_ref[...]) * dinv_ref[...] + b2_ref[...]
    m = jnp.max(z, axis=1, keepdims=True)
    zs = z - m
    o_ref[...] = zs - jnp.log(jnp.sum(jnp.exp(zs), axis=1, keepdims=True))


_deg_spec = pl.BlockSpec((_BLK, NW), lambda i: (i, 0))
_dinv_spec = pl.BlockSpec((_BLK, 1), lambda i: (i, 0))
_row_spec = pl.BlockSpec((_BLK, D), lambda i: (i, 0))
_s_spec = pl.BlockSpec((NC, _BLK, D), lambda i: (0, i, 0))
_w_spec = pl.BlockSpec((D, D), lambda i: (0, 0))
_b_spec = pl.BlockSpec((1, D), lambda i: (0, 0))

_tc1 = pl.pallas_call(
    _tc1_body,
    grid=(_GRID,),
    in_specs=[_deg_spec, _row_spec, _w_spec],
    out_specs=[_row_spec, _dinv_spec],
    out_shape=[
        jax.ShapeDtypeStruct((N, D), jnp.float32),
        jax.ShapeDtypeStruct((N, 1), jnp.float32),
    ],
)

_tc2 = pl.pallas_call(
    _tc2_body,
    grid=(_GRID,),
    in_specs=[_dinv_spec, _s_spec, _row_spec, _b_spec, _w_spec],
    out_specs=_row_spec,
    out_shape=jax.ShapeDtypeStruct((N, D), jnp.float32),
)

_tc3 = pl.pallas_call(
    _tc3_body,
    grid=(_GRID,),
    in_specs=[_dinv_spec, _s_spec, _row_spec, _b_spec],
    out_specs=_row_spec,
    out_shape=jax.ShapeDtypeStruct((N, D), jnp.float32),
)


def kernel(x, edge_index, batch, W1, b1, W2, b2):
    e4 = edge_index.reshape(2, NW, NCH, CH)
    e2 = edge_index.reshape(2, NW, EPW)
    degs = _deg_pass(e2)
    g1, dinv = _tc1(degs.T, x, W1)
    s1 = _edge_pass(g1, e4)
    g2 = _tc2(dinv, s1, g1, b1.reshape(1, D), W2)
    s2 = _edge_pass(g2, e4)
    return _tc3(dinv, s2, g2, b2.reshape(1, D))


# speedup vs baseline: 38.2356x; 1.0026x over previous
import functools

import jax
import jax.numpy as jnp
from jax import lax
from jax.experimental import pallas as pl
from jax.experimental.pallas import tpu as pltpu
from jax.experimental.pallas import tpu_sc as plsc

N = 10000
E = 320000
D = 128

NC = 2
NS = 16
NW = NC * NS
EPW = E // NW
CH = 80
NCH = EPW // CH
RPT = N // NS
ZR = 125

_mesh = plsc.VectorSubcoreMesh(core_axis_name="c", subcore_axis_name="s")
_sc_params = pltpu.CompilerParams(
    needs_layout_passes=False, use_tc_tiling_on_sc=False
)


@functools.partial(
    pl.kernel,
    mesh=_mesh,
    out_type=jax.ShapeDtypeStruct((NW, N), jnp.float32),
    compiler_params=_sc_params,
    scratch_types=[
        pltpu.VMEM((N,), jnp.float32),
        pltpu.VMEM((NCH, CH), jnp.int32),
    ],
)
def _deg_pass(e4_hbm, out_hbm, deg_l, idx_b):
    cid = lax.axis_index("c")
    sid = lax.axis_index("s")
    w = sid * NC + cid
    z16 = jnp.zeros((16,), jnp.float32)
    ones16 = jnp.ones((16,), jnp.float32)

    def zloop(i, c):
        deg_l[pl.ds(i * 16, 16)] = z16
        return c

    lax.fori_loop(0, N // 16, zloop, 0)
    pltpu.sync_copy(e4_hbm.at[1, w], idx_b)

    def cloop(i, c):
        for j in range(CH // 16):
            idx = idx_b[i, pl.ds(j * 16, 16)]
            plsc.addupdate_scatter(deg_l, [idx], ones16)
        return c

    lax.fori_loop(0, NCH, cloop, 0)
    pltpu.sync_copy(deg_l, out_hbm.at[w])


NB = 3
PK = 2


@functools.partial(
    pl.kernel,
    mesh=_mesh,
    out_type=jax.ShapeDtypeStruct((NC, N, D), jnp.float32),
    compiler_params=_sc_params,
    scratch_types=[
        pltpu.VMEM((NCH, CH), jnp.int32),
        pltpu.VMEM((NCH, CH), jnp.int32),
        pltpu.VMEM((NB, CH, D), jnp.float32),
        pltpu.VMEM_SHARED((N, D), jnp.float32),
        pltpu.SemaphoreType.DMA((NB,)),
        pltpu.SemaphoreType.DMA((NB,)),
    ],
)
def _edge_pass(g_hbm, e4_hbm, out_hbm, sidx, didx, rows, acc,
               sem_g, sem_s):
    cid = lax.axis_index("c")
    sid = lax.axis_index("s")
    w = sid * NC + cid
    z16 = jnp.zeros((16,), jnp.float32)

    pltpu.sync_copy(e4_hbm.at[0, w], sidx)
    pltpu.sync_copy(e4_hbm.at[1, w], didx)

    def zrow(i, c):
        def zcol(j, c2):
            rows[0, i, pl.ds(j * 16, 16)] = z16
            return c2
        return lax.fori_loop(0, D // 16, zcol, c)

    lax.fori_loop(0, CH, zrow, 0)
    for r in range(RPT // CH):
        pltpu.sync_copy(rows.at[0], acc.at[pl.ds(sid * RPT + r * CH, CH)])
    rem = RPT - (RPT // CH) * CH
    if rem:
        pltpu.sync_copy(
            rows.at[0, pl.ds(0, rem)],
            acc.at[pl.ds(sid * RPT + (RPT // CH) * CH, rem)],
        )
    plsc.subcore_barrier()

    for j in range(PK):
        pltpu.async_copy(g_hbm.at[sidx.at[j]], rows.at[j], sem_g.at[j])

    def eloop(i, c):
        b = lax.rem(i, NB)
        pltpu.make_async_copy(g_hbm.at[pl.ds(0, CH)], rows.at[b],
                              sem_g.at[b]).wait()
        pltpu.async_copy(rows.at[b], acc.at[didx.at[i]], sem_s.at[b], add=True)

        nxt = i + PK
        bn = lax.rem(nxt, NB)

        @pl.when(nxt < NCH)
        def _():
            @pl.when(nxt >= NB)
            def _():
                pltpu.make_async_copy(rows.at[bn], acc.at[pl.ds(0, CH)],
                                      sem_s.at[bn]).wait()
            pltpu.async_copy(g_hbm.at[sidx.at[nxt]], rows.at[bn],
                             sem_g.at[bn])
        return c

    lax.fori_loop(0, NCH, eloop, 0)
    for j in range(NB):
        pltpu.make_async_copy(rows.at[j], acc.at[pl.ds(0, CH)],
                              sem_s.at[j]).wait()
    plsc.subcore_barrier()
    pltpu.sync_copy(
        acc.at[pl.ds(sid * RPT, RPT)],
        out_hbm.at[cid, pl.ds(sid * RPT, RPT)],
    )


_BLK = 2000
_GRID = N // _BLK


def _tc1_body(degs_ref, x_ref, w_ref, g_ref, dinv_ref):
    dinv = lax.rsqrt(jnp.sum(degs_ref[...], axis=1, keepdims=True) + 1.0)
    h = jnp.dot(x_ref[...], w_ref[...], preferred_element_type=jnp.float32)
    g_ref[...] = h * dinv
    dinv_ref[...] = dinv


def _tc2_body(dinv_ref, s_ref, g1_ref, b1_ref, w2_ref, g2_ref):
    dinv = dinv_ref[...]
    s = s_ref[0] + s_ref[1] + g1_ref[...]
    h = jnp.maximum(s * dinv + b1_ref[...], 0.0)
    g2_ref[...] = jnp.dot(h, w2_ref[...], preferred_element_type=jnp.float32) * dinv


def _tc3_body(dinv_ref, s_ref, g2_ref, b2_ref, o_ref):
    z = (s_ref[0] + s_ref[1] + g2_ref[...]) * dinv_ref[...] + b2_ref[...]
    m = jnp.max(z, axis=1, keepdims=True)
    zs = z - m
    o_ref[...] = zs - jnp.log(jnp.sum(jnp.exp(zs), axis=1, keepdims=True))


_deg_spec = pl.BlockSpec((_BLK, NW), lambda i: (i, 0))
_dinv_spec = pl.BlockSpec((_BLK, 1), lambda i: (i, 0))
_row_spec = pl.BlockSpec((_BLK, D), lambda i: (i, 0))
_s_spec = pl.BlockSpec((NC, _BLK, D), lambda i: (0, i, 0))
_w_spec = pl.BlockSpec((D, D), lambda i: (0, 0))
_b_spec = pl.BlockSpec((1, D), lambda i: (0, 0))

_tc1 = pl.pallas_call(
    _tc1_body,
    grid=(_GRID,),
    in_specs=[_deg_spec, _row_spec, _w_spec],
    out_specs=[_row_spec, _dinv_spec],
    out_shape=[
        jax.ShapeDtypeStruct((N, D), jnp.float32),
        jax.ShapeDtypeStruct((N, 1), jnp.float32),
    ],
)

_tc2 = pl.pallas_call(
    _tc2_body,
    grid=(_GRID,),
    in_specs=[_dinv_spec, _s_spec, _row_spec, _b_spec, _w_spec],
    out_specs=_row_spec,
    out_shape=jax.ShapeDtypeStruct((N, D), jnp.float32),
)

_tc3 = pl.pallas_call(
    _tc3_body,
    grid=(_GRID,),
    in_specs=[_dinv_spec, _s_spec, _row_spec, _b_spec],
    out_specs=_row_spec,
    out_shape=jax.ShapeDtypeStruct((N, D), jnp.float32),
)


def kernel(x, edge_index, batch, W1, b1, W2, b2):
    e4 = edge_index.reshape(2, NW, NCH, CH)
    degs = _deg_pass(e4)
    g1, dinv = _tc1(degs.T, x, W1)
    s1 = _edge_pass(g1, e4)
    g2 = _tc2(dinv, s1, g1, b1.reshape(1, D), W2)
    s2 = _edge_pass(g2, e4)
    return _tc3(dinv, s2, g2, b2.reshape(1, D))


# speedup vs baseline: 38.6748x; 1.0115x over previous
import functools

import jax
import jax.numpy as jnp
from jax import lax
from jax.experimental import pallas as pl
from jax.experimental.pallas import tpu as pltpu
from jax.experimental.pallas import tpu_sc as plsc

N = 10000
E = 320000
D = 128

NC = 2
NS = 16
NW = NC * NS
EPW = E // NW
CH = 80
NCH = EPW // CH
RPT = N // NS
ZR = 125

_mesh = plsc.VectorSubcoreMesh(core_axis_name="c", subcore_axis_name="s")
_sc_params = pltpu.CompilerParams(
    needs_layout_passes=False, use_tc_tiling_on_sc=False
)


@functools.partial(
    pl.kernel,
    mesh=_mesh,
    out_type=jax.ShapeDtypeStruct((NW, N), jnp.float32),
    compiler_params=_sc_params,
    scratch_types=[
        pltpu.VMEM((N,), jnp.float32),
        pltpu.VMEM((NCH, CH), jnp.int32),
    ],
)
def _deg_pass(e4_hbm, out_hbm, deg_l, idx_b):
    cid = lax.axis_index("c")
    sid = lax.axis_index("s")
    w = sid * NC + cid
    z16 = jnp.zeros((16,), jnp.float32)
    ones16 = jnp.ones((16,), jnp.float32)

    def zloop(i, c):
        deg_l[pl.ds(i * 16, 16)] = z16
        return c

    lax.fori_loop(0, N // 16, zloop, 0)
    pltpu.sync_copy(e4_hbm.at[1, w], idx_b)

    def cloop(i, c):
        for j in range(CH // 16):
            idx = idx_b[i, pl.ds(j * 16, 16)]
            plsc.addupdate_scatter(deg_l, [idx], ones16)
        return c

    lax.fori_loop(0, NCH, cloop, 0)
    pltpu.sync_copy(deg_l, out_hbm.at[w])


NB = 3
PK = 2


@functools.partial(
    pl.kernel,
    mesh=_mesh,
    out_type=jax.ShapeDtypeStruct((NC, N, D), jnp.float32),
    compiler_params=_sc_params,
    scratch_types=[
        pltpu.VMEM((NCH, CH), jnp.int32),
        pltpu.VMEM((NCH, CH), jnp.int32),
        pltpu.VMEM((NB, CH, D), jnp.float32),
        pltpu.VMEM_SHARED((N, D), jnp.float32),
        pltpu.SemaphoreType.DMA((NB,)),
        pltpu.SemaphoreType.DMA((NB,)),
    ],
)
def _edge_pass(g_hbm, e4_hbm, out_hbm, sidx, didx, rows, acc,
               sem_g, sem_s):
    cid = lax.axis_index("c")
    sid = lax.axis_index("s")
    w = sid * NC + cid
    z16 = jnp.zeros((16,), jnp.float32)

    pltpu.sync_copy(e4_hbm.at[0, w], sidx)
    pltpu.sync_copy(e4_hbm.at[1, w], didx)

    def zrow(i, c):
        def zcol(j, c2):
            rows[NB - 1, i, pl.ds(j * 16, 16)] = z16
            return c2
        return lax.fori_loop(0, D // 16, zcol, c)

    lax.fori_loop(0, CH, zrow, 0)
    nfill = RPT // CH
    rem = RPT - nfill * CH
    for r in range(nfill):
        pltpu.async_copy(rows.at[NB - 1],
                         acc.at[pl.ds(sid * RPT + r * CH, CH)],
                         sem_s.at[NB - 1])
    if rem:
        pltpu.async_copy(
            rows.at[NB - 1, pl.ds(0, rem)],
            acc.at[pl.ds(sid * RPT + nfill * CH, rem)],
            sem_s.at[NB - 1],
        )

    for j in range(PK):
        pltpu.async_copy(g_hbm.at[sidx.at[j]], rows.at[j], sem_g.at[j])

    for r in range(nfill):
        pltpu.make_async_copy(rows.at[NB - 1],
                              acc.at[pl.ds(0, CH)],
                              sem_s.at[NB - 1]).wait()
    if rem:
        pltpu.make_async_copy(rows.at[NB - 1, pl.ds(0, rem)],
                              acc.at[pl.ds(0, rem)],
                              sem_s.at[NB - 1]).wait()
    plsc.subcore_barrier()

    def eloop(i, c):
        b = lax.rem(i, NB)
        pltpu.make_async_copy(g_hbm.at[pl.ds(0, CH)], rows.at[b],
                              sem_g.at[b]).wait()
        pltpu.async_copy(rows.at[b], acc.at[didx.at[i]], sem_s.at[b], add=True)

        nxt = i + PK
        bn = lax.rem(nxt, NB)

        @pl.when(nxt < NCH)
        def _():
            @pl.when(nxt >= NB)
            def _():
                pltpu.make_async_copy(rows.at[bn], acc.at[pl.ds(0, CH)],
                                      sem_s.at[bn]).wait()
            pltpu.async_copy(g_hbm.at[sidx.at[nxt]], rows.at[bn],
                             sem_g.at[bn])
        return c

    lax.fori_loop(0, NCH, eloop, 0)
    for j in range(NB):
        pltpu.make_async_copy(rows.at[j], acc.at[pl.ds(0, CH)],
                              sem_s.at[j]).wait()
    plsc.subcore_barrier()
    pltpu.sync_copy(
        acc.at[pl.ds(sid * RPT, RPT)],
        out_hbm.at[cid, pl.ds(sid * RPT, RPT)],
    )


_BLK = 2000
_GRID = N // _BLK


def _tc1_body(degs_ref, x_ref, w_ref, g_ref, dinv_ref):
    dinv = lax.rsqrt(jnp.sum(degs_ref[...], axis=1, keepdims=True) + 1.0)
    h = jnp.dot(x_ref[...], w_ref[...], preferred_element_type=jnp.float32)
    g_ref[...] = h * dinv
    dinv_ref[...] = dinv


def _tc2_body(dinv_ref, s_ref, g1_ref, b1_ref, w2_ref, g2_ref):
    dinv = dinv_ref[...]
    s = s_ref[0] + s_ref[1] + g1_ref[...]
    h = jnp.maximum(s * dinv + b1_ref[...], 0.0)
    g2_ref[...] = jnp.dot(h, w2_ref[...], preferred_element_type=jnp.float32) * dinv


def _tc3_body(dinv_ref, s_ref, g2_ref, b2_ref, o_ref):
    z = (s_ref[0] + s_ref[1] + g2_ref[...]) * dinv_ref[...] + b2_ref[...]
    m = jnp.max(z, axis=1, keepdims=True)
    zs = z - m
    o_ref[...] = zs - jnp.log(jnp.sum(jnp.exp(zs), axis=1, keepdims=True))


_deg_spec = pl.BlockSpec((_BLK, NW), lambda i: (i, 0))
_dinv_spec = pl.BlockSpec((_BLK, 1), lambda i: (i, 0))
_row_spec = pl.BlockSpec((_BLK, D), lambda i: (i, 0))
_s_spec = pl.BlockSpec((NC, _BLK, D), lambda i: (0, i, 0))
_w_spec = pl.BlockSpec((D, D), lambda i: (0, 0))
_b_spec = pl.BlockSpec((1, D), lambda i: (0, 0))

_tc1 = pl.pallas_call(
    _tc1_body,
    grid=(_GRID,),
    in_specs=[_deg_spec, _row_spec, _w_spec],
    out_specs=[_row_spec, _dinv_spec],
    out_shape=[
        jax.ShapeDtypeStruct((N, D), jnp.float32),
        jax.ShapeDtypeStruct((N, 1), jnp.float32),
    ],
)

_tc2 = pl.pallas_call(
    _tc2_body,
    grid=(_GRID,),
    in_specs=[_dinv_spec, _s_spec, _row_spec, _b_spec, _w_spec],
    out_specs=_row_spec,
    out_shape=jax.ShapeDtypeStruct((N, D), jnp.float32),
)

_tc3 = pl.pallas_call(
    _tc3_body,
    grid=(_GRID,),
    in_specs=[_dinv_spec, _s_spec, _row_spec, _b_spec],
    out_specs=_row_spec,
    out_shape=jax.ShapeDtypeStruct((N, D), jnp.float32),
)


def kernel(x, edge_index, batch, W1, b1, W2, b2):
    e4 = edge_index.reshape(2, NW, NCH, CH)
    degs = _deg_pass(e4)
    g1, dinv = _tc1(degs.T, x, W1)
    s1 = _edge_pass(g1, e4)
    g2 = _tc2(dinv, s1, g1, b1.reshape(1, D), W2)
    s2 = _edge_pass(g2, e4)
    return _tc3(dinv, s2, g2, b2.reshape(1, D))
